# Initial kernel scaffold; baseline (speedup 1.0000x reference)
import jax
import jax.numpy as jnp
from jax.experimental import pallas as pl


def kernel(x, edge_index, edge_attr, W1, b1, W2, b2, W3, b3):
    raise NotImplementedError("write your pallas kernel here")


# speedup vs baseline: 8.3911x; 8.3911x over previous
import functools

import jax
import jax.numpy as jnp
from jax import lax
from jax.experimental import pallas as pl
from jax.experimental.pallas import tpu as pltpu
from jax.experimental.pallas import tpu_sc as plsc

N = 10000
E = 320000
D = 128

NC = 2
NS = 16
NW = NC * NS
EPT = E // NW
CH = 80
NCHUNK = EPT // CH
NP = 10240
RPT = NP // NS
RB = 1000

_mesh = plsc.VectorSubcoreMesh(core_axis_name="c", subcore_axis_name="s")


def _zero_rows(buf, n_rows, width):
    z = jnp.zeros((16,), jnp.float32)

    def body(j, _):
        for q in range(width // 16):
            buf[j, pl.ds(q * 16, 16)] = z
        return 0

    lax.fori_loop(0, n_rows, body, 0)


def _sc_deg_body(col_hbm, w_hbm, out_hbm, acc_sh, col_c, w_c, msg_v):
    cid = lax.axis_index("c")
    sid = lax.axis_index("s")
    wid = sid * NC + cid

    _zero_rows(msg_v, CH, 16)
    for r in range(RPT // CH):
        pltpu.sync_copy(msg_v, acc_sh.at[pl.ds(sid * RPT + r * CH, CH)])
    plsc.subcore_barrier()

    def chunk(k, _):
        base = wid * EPT + k * CH
        pltpu.sync_copy(col_hbm.at[pl.ds(base, CH)], col_c)
        pltpu.sync_copy(w_hbm.at[pl.ds(base, CH)], w_c)

        def group(g, _):
            wv = w_c[pl.ds(g * 16, 16)]
            for l in range(16):
                msg_v[g * 16 + l, pl.ds(0, 16)] = jnp.full((16,), wv[l],
                                                           jnp.float32)
            return 0

        lax.fori_loop(0, CH // 16, group, 0)
        pltpu.sync_copy(msg_v, acc_sh.at[col_c], add=True)
        return 0

    lax.fori_loop(0, NCHUNK, chunk, 0)
    plsc.subcore_barrier()
    pltpu.sync_copy(acc_sh.at[pl.ds(sid * RPT, RPT)],
                    out_hbm.at[cid, pl.ds(sid * RPT, RPT)])


_sc_deg = pl.kernel(
    _sc_deg_body,
    out_type=jax.ShapeDtypeStruct((NC, NP, 16), jnp.float32),
    mesh=_mesh,
    scratch_types=[
        pltpu.VMEM_SHARED((NP, 16), jnp.float32),
        pltpu.VMEM((CH,), jnp.int32),
        pltpu.VMEM((CH,), jnp.float32),
        pltpu.VMEM((CH, 16), jnp.float32),
    ],
)


def _sc_layer_body(g_hbm, row_hbm, col_hbm, w_hbm, out_hbm,
                   acc_sh, row_c, col_c, w_c, rows_v):
    cid = lax.axis_index("c")
    sid = lax.axis_index("s")
    wid = sid * NC + cid

    _zero_rows(rows_v, CH, D)
    for r in range(RPT // CH):
        pltpu.sync_copy(rows_v, acc_sh.at[pl.ds(sid * RPT + r * CH, CH)])
    plsc.subcore_barrier()

    def chunk(k, _):
        base = wid * EPT + k * CH
        pltpu.sync_copy(row_hbm.at[pl.ds(base, CH)], row_c)
        pltpu.sync_copy(col_hbm.at[pl.ds(base, CH)], col_c)
        pltpu.sync_copy(w_hbm.at[pl.ds(base, CH)], w_c)
        pltpu.sync_copy(g_hbm.at[row_c], rows_v)

        def group(g, _):
            wv = w_c[pl.ds(g * 16, 16)]
            for l in range(16):
                wb = jnp.full((16,), wv[l], jnp.float32)
                j = g * 16 + l
                for q in range(D // 16):
                    sl = pl.ds(q * 16, 16)
                    rows_v[j, sl] = rows_v[j, sl] * wb
            return 0

        lax.fori_loop(0, CH // 16, group, 0)
        pltpu.sync_copy(rows_v, acc_sh.at[col_c], add=True)
        return 0

    lax.fori_loop(0, NCHUNK, chunk, 0)
    plsc.subcore_barrier()
    pltpu.sync_copy(acc_sh.at[pl.ds(sid * RPT, RPT)],
                    out_hbm.at[cid, pl.ds(sid * RPT, RPT)])


_sc_layer = pl.kernel(
    _sc_layer_body,
    out_type=jax.ShapeDtypeStruct((NC, NP, D), jnp.float32),
    mesh=_mesh,
    scratch_types=[
        pltpu.VMEM_SHARED((NP, D), jnp.float32),
        pltpu.VMEM((CH,), jnp.int32),
        pltpu.VMEM((CH,), jnp.int32),
        pltpu.VMEM((CH,), jnp.float32),
        pltpu.VMEM((CH, D), jnp.float32),
    ],
)


def _tc1_body(d_ref, x_ref, w_ref, a_ref, g_ref, dis_ref, dinv_ref):
    d = d_ref[...]
    deg = d[0, :, 0] + d[1, :, 0] + 1.0
    dis = lax.rsqrt(deg)
    dinv = 1.0 / deg
    a = jnp.dot(x_ref[...], w_ref[...], preferred_element_type=jnp.float32)
    a_ref[...] = a
    g_ref[...] = a * dis[:, None]
    dis_ref[...] = dis[:, None]
    dinv_ref[...] = dinv[:, None]


def _tc1(deg_parts, x, W1):
    return pl.pallas_call(
        _tc1_body,
        grid=(N // RB,),
        in_specs=[
            pl.BlockSpec((NC, RB, 16), lambda i: (0, i, 0)),
            pl.BlockSpec((RB, D), lambda i: (i, 0)),
            pl.BlockSpec((D, D), lambda i: (0, 0)),
        ],
        out_specs=[
            pl.BlockSpec((RB, D), lambda i: (i, 0)),
            pl.BlockSpec((RB, D), lambda i: (i, 0)),
            pl.BlockSpec((RB, 1), lambda i: (i, 0)),
            pl.BlockSpec((RB, 1), lambda i: (i, 0)),
        ],
        out_shape=[
            jax.ShapeDtypeStruct((N, D), jnp.float32),
            jax.ShapeDtypeStruct((N, D), jnp.float32),
            jax.ShapeDtypeStruct((N, 1), jnp.float32),
            jax.ShapeDtypeStruct((N, 1), jnp.float32),
        ],
    )(deg_parts, x, W1)


def _tc_mid_body(s_ref, a_ref, dis_ref, dinv_ref, b_ref, w_ref,
                 an_ref, gn_ref):
    s = s_ref[0] + s_ref[1]
    dis = dis_ref[...]
    z = dis * s + dinv_ref[...] * a_ref[...] + b_ref[...]
    x2 = jnp.maximum(z, 0.0)
    a2 = jnp.dot(x2, w_ref[...], preferred_element_type=jnp.float32)
    an_ref[...] = a2
    gn_ref[...] = a2 * dis


def _tc_mid(s, a_prev, dis, dinv, b, W):
    return pl.pallas_call(
        _tc_mid_body,
        grid=(N // RB,),
        in_specs=[
            pl.BlockSpec((NC, RB, D), lambda i: (0, i, 0)),
            pl.BlockSpec((RB, D), lambda i: (i, 0)),
            pl.BlockSpec((RB, 1), lambda i: (i, 0)),
            pl.BlockSpec((RB, 1), lambda i: (i, 0)),
            pl.BlockSpec((1, D), lambda i: (0, 0)),
            pl.BlockSpec((D, D), lambda i: (0, 0)),
        ],
        out_specs=[
            pl.BlockSpec((RB, D), lambda i: (i, 0)),
            pl.BlockSpec((RB, D), lambda i: (i, 0)),
        ],
        out_shape=[
            jax.ShapeDtypeStruct((N, D), jnp.float32),
            jax.ShapeDtypeStruct((N, D), jnp.float32),
        ],
    )(s, a_prev, dis, dinv, b, W)


def _tc_final_body(s_ref, a_ref, dis_ref, dinv_ref, b_ref, o_ref):
    s = s_ref[0] + s_ref[1]
    z = dis_ref[...] * s + dinv_ref[...] * a_ref[...] + b_ref[...]
    m = jnp.max(z, axis=1, keepdims=True)
    lse = jnp.log(jnp.sum(jnp.exp(z - m), axis=1, keepdims=True)) + m
    o_ref[...] = z - lse


def _tc_final(s, a_prev, dis, dinv, b):
    return pl.pallas_call(
        _tc_final_body,
        grid=(N // RB,),
        in_specs=[
            pl.BlockSpec((NC, RB, D), lambda i: (0, i, 0)),
            pl.BlockSpec((RB, D), lambda i: (i, 0)),
            pl.BlockSpec((RB, 1), lambda i: (i, 0)),
            pl.BlockSpec((RB, 1), lambda i: (i, 0)),
            pl.BlockSpec((1, D), lambda i: (0, 0)),
        ],
        out_specs=pl.BlockSpec((RB, D), lambda i: (i, 0)),
        out_shape=jax.ShapeDtypeStruct((N, D), jnp.float32),
    )(s, a_prev, dis, dinv, b)


@jax.jit
def kernel(x, edge_index, edge_attr, W1, b1, W2, b2, W3, b3):
    row = edge_index[0]
    col = edge_index[1]
    w_t = edge_attr

    deg_parts = _sc_deg(col, w_t)
    a1, g1, dis, dinv = _tc1(deg_parts, x, W1)
    s1 = _sc_layer(g1, row, col, w_t)
    a2, g2 = _tc_mid(s1, a1, dis, dinv, b1.reshape(1, D), W2)
    s2 = _sc_layer(g2, row, col, w_t)
    a3, g3 = _tc_mid(s2, a2, dis, dinv, b2.reshape(1, D), W3)
    s3 = _sc_layer(g3, row, col, w_t)
    return _tc_final(s3, a3, dis, dinv, b3.reshape(1, D))


# speedup vs baseline: 14.0862x; 1.6787x over previous
import functools

import jax
import jax.numpy as jnp
from jax import lax
from jax.experimental import pallas as pl
from jax.experimental.pallas import tpu as pltpu
from jax.experimental.pallas import tpu_sc as plsc

N = 10000
E = 320000
D = 128

NC = 2
NS = 16
NW = NC * NS
EPT = E // NW
CH = 80
NCHUNK = EPT // CH
NP = 10240
RPT = NP // NS
RB = 1000

_mesh = plsc.VectorSubcoreMesh(core_axis_name="c", subcore_axis_name="s")


def _zero_rows(buf, n_rows, width):
    z = jnp.zeros((16,), jnp.float32)

    def body(j, _):
        for q in range(width // 16):
            buf[j, pl.ds(q * 16, 16)] = z
        return 0

    lax.fori_loop(0, n_rows, body, 0)


def _sc_deg_body(col_hbm, w_hbm, out_hbm, acc_sh, col_c, w_c, msg_v):
    cid = lax.axis_index("c")
    sid = lax.axis_index("s")
    wid = sid * NC + cid

    _zero_rows(msg_v, CH, 16)
    for r in range(RPT // CH):
        pltpu.sync_copy(msg_v, acc_sh.at[pl.ds(sid * RPT + r * CH, CH)])
    plsc.subcore_barrier()

    def chunk(k, _):
        base = wid * EPT + k * CH
        pltpu.sync_copy(col_hbm.at[pl.ds(base, CH)], col_c)
        pltpu.sync_copy(w_hbm.at[pl.ds(base, CH)], w_c)

        def group(g, _):
            wv = w_c[pl.ds(g * 16, 16)]
            for l in range(16):
                msg_v[g * 16 + l, pl.ds(0, 16)] = jnp.full((16,), wv[l],
                                                           jnp.float32)
            return 0

        lax.fori_loop(0, CH // 16, group, 0)
        pltpu.sync_copy(msg_v, acc_sh.at[col_c], add=True)
        return 0

    lax.fori_loop(0, NCHUNK, chunk, 0)
    plsc.subcore_barrier()
    pltpu.sync_copy(acc_sh.at[pl.ds(sid * RPT, RPT)],
                    out_hbm.at[cid, pl.ds(sid * RPT, RPT)])


_sc_deg = pl.kernel(
    _sc_deg_body,
    out_type=jax.ShapeDtypeStruct((NC, NP, 16), jnp.float32),
    mesh=_mesh,
    scratch_types=[
        pltpu.VMEM_SHARED((NP, 16), jnp.float32),
        pltpu.VMEM((CH,), jnp.int32),
        pltpu.VMEM((CH,), jnp.float32),
        pltpu.VMEM((CH, 16), jnp.float32),
    ],
)


NB = 3


def _sc_layer_body(g_hbm, comb_hbm, w_hbm, out_hbm,
                   acc_sh, comb_v, w_v, rows_v, *sems):
    gsem = sems[0:NB]
    ssem = sems[NB:2 * NB]
    isem = sems[2 * NB:3 * NB]
    cid = lax.axis_index("c")
    sid = lax.axis_index("s")
    wid = sid * NC + cid

    _zero_rows(rows_v.at[0], CH, D)
    for r in range(RPT // CH):
        pltpu.sync_copy(rows_v.at[0], acc_sh.at[pl.ds(sid * RPT + r * CH, CH)])
    plsc.subcore_barrier()

    def scale(b, c):
        def group(g, _):
            wv = w_v[b, 0, pl.ds(g * 16, 16)]
            for l in range(16):
                wb = jnp.full((16,), wv[l], jnp.float32)
                j = g * 16 + l
                for q in range(D // 16):
                    sl = pl.ds(q * 16, 16)
                    rows_v[b, j, sl] = rows_v[b, j, sl] * wb
            return 0

        lax.fori_loop(0, CH // 16, group, 0)

    def triple(t, _):
        cs = [NB * t + i for i in range(NB)]
        fills = []
        for i, c in enumerate(cs):
            fills.append(
                (pltpu.async_copy(comb_hbm.at[wid, c], comb_v.at[i], isem[i]),
                 pltpu.async_copy(w_hbm.at[wid, c], w_v.at[i], isem[i])))
        for i in range(NB):
            fills[i][0].wait()
            fills[i][1].wait()
        gd = pltpu.async_copy(g_hbm.at[comb_v.at[0, 0]], rows_v.at[0],
                              gsem[0])
        gd.wait()
        for i, c in enumerate(cs):
            if i + 1 < NB:
                gd = pltpu.async_copy(g_hbm.at[comb_v.at[i + 1, 0]],
                                      rows_v.at[i + 1], gsem[i + 1])
            scale(i, c)
            pltpu.async_copy(rows_v.at[i], acc_sh.at[comb_v.at[i, 1]],
                             ssem[i], add=True).wait()
            if i + 1 < NB:
                gd.wait()
        return 0

    lax.fori_loop(0, NCHUNK // NB, triple, 0)

    for c in range(NCHUNK - NCHUNK % NB, NCHUNK):
        pltpu.async_copy(comb_hbm.at[wid, c], comb_v.at[0], isem[0]).wait()
        pltpu.async_copy(w_hbm.at[wid, c], w_v.at[0], isem[0]).wait()
        pltpu.async_copy(g_hbm.at[comb_v.at[0, 0]], rows_v.at[0],
                         gsem[0]).wait()
        scale(0, c)
        pltpu.async_copy(rows_v.at[0], acc_sh.at[comb_v.at[0, 1]],
                         ssem[0], add=True).wait()

    plsc.subcore_barrier()
    pltpu.sync_copy(acc_sh.at[pl.ds(sid * RPT, RPT)],
                    out_hbm.at[cid, pl.ds(sid * RPT, RPT)])


_sc_layer = pl.kernel(
    _sc_layer_body,
    out_type=jax.ShapeDtypeStruct((NC, NP, D), jnp.float32),
    mesh=_mesh,
    scratch_types=[
        pltpu.VMEM_SHARED((NP, D), jnp.float32),
        pltpu.VMEM((NB, 2, CH), jnp.int32),
        pltpu.VMEM((NB, 1, CH), jnp.float32),
        pltpu.VMEM((NB, CH, D), jnp.float32),
    ] + [pltpu.SemaphoreType.DMA] * (3 * NB),
)


def _tc1_body(d_ref, x_ref, w_ref, a_ref, g_ref, dis_ref, dinv_ref):
    d = d_ref[...]
    deg = d[0, :, 0] + d[1, :, 0] + 1.0
    dis = lax.rsqrt(deg)
    dinv = 1.0 / deg
    a = jnp.dot(x_ref[...], w_ref[...], preferred_element_type=jnp.float32)
    a_ref[...] = a
    g_ref[...] = a * dis[:, None]
    dis_ref[...] = dis[:, None]
    dinv_ref[...] = dinv[:, None]


def _tc1(deg_parts, x, W1):
    return pl.pallas_call(
        _tc1_body,
        grid=(N // RB,),
        in_specs=[
            pl.BlockSpec((NC, RB, 16), lambda i: (0, i, 0)),
            pl.BlockSpec((RB, D), lambda i: (i, 0)),
            pl.BlockSpec((D, D), lambda i: (0, 0)),
        ],
        out_specs=[
            pl.BlockSpec((RB, D), lambda i: (i, 0)),
            pl.BlockSpec((RB, D), lambda i: (i, 0)),
            pl.BlockSpec((RB, 1), lambda i: (i, 0)),
            pl.BlockSpec((RB, 1), lambda i: (i, 0)),
        ],
        out_shape=[
            jax.ShapeDtypeStruct((N, D), jnp.float32),
            jax.ShapeDtypeStruct((N, D), jnp.float32),
            jax.ShapeDtypeStruct((N, 1), jnp.float32),
            jax.ShapeDtypeStruct((N, 1), jnp.float32),
        ],
    )(deg_parts, x, W1)


def _tc_mid_body(s_ref, a_ref, dis_ref, dinv_ref, b_ref, w_ref,
                 an_ref, gn_ref):
    s = s_ref[0] + s_ref[1]
    dis = dis_ref[...]
    z = dis * s + dinv_ref[...] * a_ref[...] + b_ref[...]
    x2 = jnp.maximum(z, 0.0)
    a2 = jnp.dot(x2, w_ref[...], preferred_element_type=jnp.float32)
    an_ref[...] = a2
    gn_ref[...] = a2 * dis


def _tc_mid(s, a_prev, dis, dinv, b, W):
    return pl.pallas_call(
        _tc_mid_body,
        grid=(N // RB,),
        in_specs=[
            pl.BlockSpec((NC, RB, D), lambda i: (0, i, 0)),
            pl.BlockSpec((RB, D), lambda i: (i, 0)),
            pl.BlockSpec((RB, 1), lambda i: (i, 0)),
            pl.BlockSpec((RB, 1), lambda i: (i, 0)),
            pl.BlockSpec((1, D), lambda i: (0, 0)),
            pl.BlockSpec((D, D), lambda i: (0, 0)),
        ],
        out_specs=[
            pl.BlockSpec((RB, D), lambda i: (i, 0)),
            pl.BlockSpec((RB, D), lambda i: (i, 0)),
        ],
        out_shape=[
            jax.ShapeDtypeStruct((N, D), jnp.float32),
            jax.ShapeDtypeStruct((N, D), jnp.float32),
        ],
    )(s, a_prev, dis, dinv, b, W)


def _tc_final_body(s_ref, a_ref, dis_ref, dinv_ref, b_ref, o_ref):
    s = s_ref[0] + s_ref[1]
    z = dis_ref[...] * s + dinv_ref[...] * a_ref[...] + b_ref[...]
    m = jnp.max(z, axis=1, keepdims=True)
    lse = jnp.log(jnp.sum(jnp.exp(z - m), axis=1, keepdims=True)) + m
    o_ref[...] = z - lse


def _tc_final(s, a_prev, dis, dinv, b):
    return pl.pallas_call(
        _tc_final_body,
        grid=(N // RB,),
        in_specs=[
            pl.BlockSpec((NC, RB, D), lambda i: (0, i, 0)),
            pl.BlockSpec((RB, D), lambda i: (i, 0)),
            pl.BlockSpec((RB, 1), lambda i: (i, 0)),
            pl.BlockSpec((RB, 1), lambda i: (i, 0)),
            pl.BlockSpec((1, D), lambda i: (0, 0)),
        ],
        out_specs=pl.BlockSpec((RB, D), lambda i: (i, 0)),
        out_shape=jax.ShapeDtypeStruct((N, D), jnp.float32),
    )(s, a_prev, dis, dinv, b)


@jax.jit
def kernel(x, edge_index, edge_attr, W1, b1, W2, b2, W3, b3):
    col = edge_index[1]
    w_t = edge_attr
    row_r = edge_index[0].reshape(NW, NCHUNK, 1, CH)
    col_r = col.reshape(NW, NCHUNK, 1, CH)
    comb = jnp.concatenate([row_r, col_r], axis=2)
    w_r = w_t.reshape(NW, NCHUNK, 1, CH)

    deg_parts = _sc_deg(col, w_t)
    a1, g1, dis, dinv = _tc1(deg_parts, x, W1)
    s1 = _sc_layer(g1, comb, w_r)
    a2, g2 = _tc_mid(s1, a1, dis, dinv, b1.reshape(1, D), W2)
    s2 = _sc_layer(g2, comb, w_r)
    a3, g3 = _tc_mid(s2, a2, dis, dinv, b2.reshape(1, D), W3)
    s3 = _sc_layer(g3, comb, w_r)
    return _tc_final(s3, a3, dis, dinv, b3.reshape(1, D))


# speedup vs baseline: 15.2504x; 1.0827x over previous
import functools

import jax
import jax.numpy as jnp
from jax import lax
from jax.experimental import pallas as pl
from jax.experimental.pallas import tpu as pltpu
from jax.experimental.pallas import tpu_sc as plsc

N = 10000
E = 320000
D = 128

NC = 2
NS = 16
NW = NC * NS
EPT = E // NW
CH = 80
NCHUNK = EPT // CH
NP = 10240
RPT = NP // NS
RB = 1000

_mesh = plsc.VectorSubcoreMesh(core_axis_name="c", subcore_axis_name="s")


def _zero_rows(buf, n_rows, width):
    z = jnp.zeros((16,), jnp.float32)

    def body(j, _):
        for q in range(width // 16):
            buf[j, pl.ds(q * 16, 16)] = z
        return 0

    lax.fori_loop(0, n_rows, body, 0)


NB = 3


def _sc_deg_body(comb_hbm, w_hbm, out_hbm, acc_sh, comb_v, w_v, msg_v, *sems):
    ssem = sems[0:NB]
    isem = sems[NB:2 * NB]
    cid = lax.axis_index("c")
    sid = lax.axis_index("s")
    wid = sid * NC + cid

    _zero_rows(msg_v.at[0], CH, 16)
    for r in range(RPT // CH):
        pltpu.sync_copy(msg_v.at[0], acc_sh.at[pl.ds(sid * RPT + r * CH, CH)])
    plsc.subcore_barrier()

    def build(i):
        def group(g, _):
            wv = w_v[i, 0, pl.ds(g * 16, 16)]
            for l in range(16):
                msg_v[i, g * 16 + l, pl.ds(0, 16)] = jnp.full((16,), wv[l],
                                                              jnp.float32)
            return 0

        lax.fori_loop(0, CH // 16, group, 0)

    def triple(t, _):
        cs = [NB * t + i for i in range(NB)]
        fills = []
        for i, c in enumerate(cs):
            fills.append(
                (pltpu.async_copy(comb_hbm.at[wid, c], comb_v.at[i], isem[i]),
                 pltpu.async_copy(w_hbm.at[wid, c], w_v.at[i], isem[i])))
        sd = None
        for i, c in enumerate(cs):
            fills[i][0].wait()
            fills[i][1].wait()
            build(i)
            if sd is not None:
                sd.wait()
            sd = pltpu.async_copy(msg_v.at[i], acc_sh.at[comb_v.at[i, 1]],
                                  ssem[i], add=True)
        sd.wait()
        return 0

    lax.fori_loop(0, NCHUNK // NB, triple, 0)

    for c in range(NCHUNK - NCHUNK % NB, NCHUNK):
        pltpu.async_copy(comb_hbm.at[wid, c], comb_v.at[0], isem[0]).wait()
        pltpu.async_copy(w_hbm.at[wid, c], w_v.at[0], isem[0]).wait()
        build(0)
        pltpu.async_copy(msg_v.at[0], acc_sh.at[comb_v.at[0, 1]],
                         ssem[0], add=True).wait()

    plsc.subcore_barrier()
    pltpu.sync_copy(acc_sh.at[pl.ds(sid * RPT, RPT)],
                    out_hbm.at[cid, pl.ds(sid * RPT, RPT)])


_sc_deg = pl.kernel(
    _sc_deg_body,
    out_type=jax.ShapeDtypeStruct((NC, NP, 16), jnp.float32),
    mesh=_mesh,
    scratch_types=[
        pltpu.VMEM_SHARED((NP, 16), jnp.float32),
        pltpu.VMEM((NB, 2, CH), jnp.int32),
        pltpu.VMEM((NB, 1, CH), jnp.float32),
        pltpu.VMEM((NB, CH, 16), jnp.float32),
    ] + [pltpu.SemaphoreType.DMA] * (2 * NB),
)


def _sc_layer_body(g_hbm, comb_hbm, w_hbm, out_hbm,
                   acc_sh, comb_v, w_v, rows_v, *sems):
    gsem = sems[0:NB]
    ssem = sems[NB:2 * NB]
    isem = sems[2 * NB:3 * NB]
    cid = lax.axis_index("c")
    sid = lax.axis_index("s")
    wid = sid * NC + cid

    _zero_rows(rows_v.at[0], CH, D)
    for r in range(RPT // CH):
        pltpu.sync_copy(rows_v.at[0], acc_sh.at[pl.ds(sid * RPT + r * CH, CH)])
    plsc.subcore_barrier()

    def scale(b, c):
        def group(g, _):
            wv = w_v[b, 0, pl.ds(g * 16, 16)]
            for l in range(16):
                wb = jnp.full((16,), wv[l], jnp.float32)
                j = g * 16 + l
                for q in range(D // 16):
                    sl = pl.ds(q * 16, 16)
                    rows_v[b, j, sl] = rows_v[b, j, sl] * wb
            return 0

        lax.fori_loop(0, CH // 16, group, 0)

    def triple(t, _):
        cs = [NB * t + i for i in range(NB)]
        fills = []
        for i, c in enumerate(cs):
            fills.append(
                (pltpu.async_copy(comb_hbm.at[wid, c], comb_v.at[i], isem[i]),
                 pltpu.async_copy(w_hbm.at[wid, c], w_v.at[i], isem[i])))
        for i in range(NB):
            fills[i][0].wait()
            fills[i][1].wait()
        gd = pltpu.async_copy(g_hbm.at[comb_v.at[0, 0]], rows_v.at[0],
                              gsem[0])
        gd.wait()
        sd = None
        for i, c in enumerate(cs):
            if i + 1 < NB:
                gd = pltpu.async_copy(g_hbm.at[comb_v.at[i + 1, 0]],
                                      rows_v.at[i + 1], gsem[i + 1])
            scale(i, c)
            if sd is not None:
                sd.wait()
            sd = pltpu.async_copy(rows_v.at[i], acc_sh.at[comb_v.at[i, 1]],
                                  ssem[i], add=True)
            if i + 1 < NB:
                gd.wait()
        sd.wait()
        return 0

    lax.fori_loop(0, NCHUNK // NB, triple, 0)

    for c in range(NCHUNK - NCHUNK % NB, NCHUNK):
        pltpu.async_copy(comb_hbm.at[wid, c], comb_v.at[0], isem[0]).wait()
        pltpu.async_copy(w_hbm.at[wid, c], w_v.at[0], isem[0]).wait()
        pltpu.async_copy(g_hbm.at[comb_v.at[0, 0]], rows_v.at[0],
                         gsem[0]).wait()
        scale(0, c)
        pltpu.async_copy(rows_v.at[0], acc_sh.at[comb_v.at[0, 1]],
                         ssem[0], add=True).wait()

    plsc.subcore_barrier()
    pltpu.sync_copy(acc_sh.at[pl.ds(sid * RPT, RPT)],
                    out_hbm.at[cid, pl.ds(sid * RPT, RPT)])


_sc_layer = pl.kernel(
    _sc_layer_body,
    out_type=jax.ShapeDtypeStruct((NC, NP, D), jnp.float32),
    mesh=_mesh,
    scratch_types=[
        pltpu.VMEM_SHARED((NP, D), jnp.float32),
        pltpu.VMEM((NB, 2, CH), jnp.int32),
        pltpu.VMEM((NB, 1, CH), jnp.float32),
        pltpu.VMEM((NB, CH, D), jnp.float32),
    ] + [pltpu.SemaphoreType.DMA] * (3 * NB),
)


def _tc1_body(d_ref, x_ref, w_ref, a_ref, g_ref, dis_ref, dinv_ref):
    d = d_ref[...]
    deg = d[0, :, 0] + d[1, :, 0] + 1.0
    dis = lax.rsqrt(deg)
    dinv = 1.0 / deg
    a = jnp.dot(x_ref[...], w_ref[...], preferred_element_type=jnp.float32)
    a_ref[...] = a
    g_ref[...] = a * dis[:, None]
    dis_ref[...] = dis[:, None]
    dinv_ref[...] = dinv[:, None]


def _tc1(deg_parts, x, W1):
    return pl.pallas_call(
        _tc1_body,
        grid=(N // RB,),
        in_specs=[
            pl.BlockSpec((NC, RB, 16), lambda i: (0, i, 0)),
            pl.BlockSpec((RB, D), lambda i: (i, 0)),
            pl.BlockSpec((D, D), lambda i: (0, 0)),
        ],
        out_specs=[
            pl.BlockSpec((RB, D), lambda i: (i, 0)),
            pl.BlockSpec((RB, D), lambda i: (i, 0)),
            pl.BlockSpec((RB, 1), lambda i: (i, 0)),
            pl.BlockSpec((RB, 1), lambda i: (i, 0)),
        ],
        out_shape=[
            jax.ShapeDtypeStruct((N, D), jnp.float32),
            jax.ShapeDtypeStruct((N, D), jnp.float32),
            jax.ShapeDtypeStruct((N, 1), jnp.float32),
            jax.ShapeDtypeStruct((N, 1), jnp.float32),
        ],
    )(deg_parts, x, W1)


def _tc_mid_body(s_ref, a_ref, dis_ref, dinv_ref, b_ref, w_ref,
                 an_ref, gn_ref):
    s = s_ref[0] + s_ref[1]
    dis = dis_ref[...]
    z = dis * s + dinv_ref[...] * a_ref[...] + b_ref[...]
    x2 = jnp.maximum(z, 0.0)
    a2 = jnp.dot(x2, w_ref[...], preferred_element_type=jnp.float32)
    an_ref[...] = a2
    gn_ref[...] = a2 * dis


def _tc_mid(s, a_prev, dis, dinv, b, W):
    return pl.pallas_call(
        _tc_mid_body,
        grid=(N // RB,),
        in_specs=[
            pl.BlockSpec((NC, RB, D), lambda i: (0, i, 0)),
            pl.BlockSpec((RB, D), lambda i: (i, 0)),
            pl.BlockSpec((RB, 1), lambda i: (i, 0)),
            pl.BlockSpec((RB, 1), lambda i: (i, 0)),
            pl.BlockSpec((1, D), lambda i: (0, 0)),
            pl.BlockSpec((D, D), lambda i: (0, 0)),
        ],
        out_specs=[
            pl.BlockSpec((RB, D), lambda i: (i, 0)),
            pl.BlockSpec((RB, D), lambda i: (i, 0)),
        ],
        out_shape=[
            jax.ShapeDtypeStruct((N, D), jnp.float32),
            jax.ShapeDtypeStruct((N, D), jnp.float32),
        ],
    )(s, a_prev, dis, dinv, b, W)


def _tc_final_body(s_ref, a_ref, dis_ref, dinv_ref, b_ref, o_ref):
    s = s_ref[0] + s_ref[1]
    z = dis_ref[...] * s + dinv_ref[...] * a_ref[...] + b_ref[...]
    m = jnp.max(z, axis=1, keepdims=True)
    lse = jnp.log(jnp.sum(jnp.exp(z - m), axis=1, keepdims=True)) + m
    o_ref[...] = z - lse


def _tc_final(s, a_prev, dis, dinv, b):
    return pl.pallas_call(
        _tc_final_body,
        grid=(N // RB,),
        in_specs=[
            pl.BlockSpec((NC, RB, D), lambda i: (0, i, 0)),
            pl.BlockSpec((RB, D), lambda i: (i, 0)),
            pl.BlockSpec((RB, 1), lambda i: (i, 0)),
            pl.BlockSpec((RB, 1), lambda i: (i, 0)),
            pl.BlockSpec((1, D), lambda i: (0, 0)),
        ],
        out_specs=pl.BlockSpec((RB, D), lambda i: (i, 0)),
        out_shape=jax.ShapeDtypeStruct((N, D), jnp.float32),
    )(s, a_prev, dis, dinv, b)


@jax.jit
def kernel(x, edge_index, edge_attr, W1, b1, W2, b2, W3, b3):
    col = edge_index[1]
    w_t = edge_attr
    row_r = edge_index[0].reshape(NW, NCHUNK, 1, CH)
    col_r = col.reshape(NW, NCHUNK, 1, CH)
    comb = jnp.concatenate([row_r, col_r], axis=2)
    w_r = w_t.reshape(NW, NCHUNK, 1, CH)

    deg_parts = _sc_deg(comb, w_r)
    a1, g1, dis, dinv = _tc1(deg_parts, x, W1)
    s1 = _sc_layer(g1, comb, w_r)
    a2, g2 = _tc_mid(s1, a1, dis, dinv, b1.reshape(1, D), W2)
    s2 = _sc_layer(g2, comb, w_r)
    a3, g3 = _tc_mid(s2, a2, dis, dinv, b2.reshape(1, D), W3)
    s3 = _sc_layer(g3, comb, w_r)
    return _tc_final(s3, a3, dis, dinv, b3.reshape(1, D))


# speedup vs baseline: 16.6125x; 1.0893x over previous
import functools

import jax
import jax.numpy as jnp
from jax import lax
from jax.experimental import pallas as pl
from jax.experimental.pallas import tpu as pltpu
from jax.experimental.pallas import tpu_sc as plsc

N = 10000
E = 320000
D = 128

NC = 2
NS = 16
NW = NC * NS
EPT = E // NW
CH = 80
NCHUNK = EPT // CH
NP = 10240
RPT = NP // NS
RB = 1000

_mesh = plsc.VectorSubcoreMesh(core_axis_name="c", subcore_axis_name="s")


def _zero_rows(buf, n_rows, width):
    z = jnp.zeros((16,), jnp.float32)

    def body(j, _):
        for q in range(width // 16):
            buf[j, pl.ds(q * 16, 16)] = z
        return 0

    lax.fori_loop(0, n_rows, body, 0)


NB = 3


def _sc_deg_body(comb_hbm, w_hbm, out_hbm, acc_sh, comb_v, w_v, msg_v, *sems):
    ssem = sems[0:NB]
    isem = sems[NB:2 * NB]
    cid = lax.axis_index("c")
    sid = lax.axis_index("s")
    wid = sid * NC + cid

    _zero_rows(msg_v.at[0], CH, 16)
    for r in range(RPT // CH):
        pltpu.sync_copy(msg_v.at[0], acc_sh.at[pl.ds(sid * RPT + r * CH, CH)])
    plsc.subcore_barrier()

    def build(i):
        def group(g, _):
            wv = w_v[i, 0, pl.ds(g * 16, 16)]
            for l in range(16):
                msg_v[i, g * 16 + l, pl.ds(0, 16)] = jnp.full((16,), wv[l],
                                                              jnp.float32)
            return 0

        lax.fori_loop(0, CH // 16, group, 0)

    def triple(t, _):
        cs = [NB * t + i for i in range(NB)]
        fills = []
        for i, c in enumerate(cs):
            fills.append(
                (pltpu.async_copy(comb_hbm.at[wid, c], comb_v.at[i], isem[i]),
                 pltpu.async_copy(w_hbm.at[wid, c], w_v.at[i], isem[i])))
        sd = None
        for i, c in enumerate(cs):
            fills[i][0].wait()
            fills[i][1].wait()
            build(i)
            if sd is not None:
                sd.wait()
            sd = pltpu.async_copy(msg_v.at[i], acc_sh.at[comb_v.at[i, 1]],
                                  ssem[i], add=True)
        sd.wait()
        return 0

    lax.fori_loop(0, NCHUNK // NB, triple, 0)

    for c in range(NCHUNK - NCHUNK % NB, NCHUNK):
        pltpu.async_copy(comb_hbm.at[wid, c], comb_v.at[0], isem[0]).wait()
        pltpu.async_copy(w_hbm.at[wid, c], w_v.at[0], isem[0]).wait()
        build(0)
        pltpu.async_copy(msg_v.at[0], acc_sh.at[comb_v.at[0, 1]],
                         ssem[0], add=True).wait()

    plsc.subcore_barrier()
    pltpu.sync_copy(acc_sh.at[pl.ds(sid * RPT, RPT)],
                    out_hbm.at[cid, pl.ds(sid * RPT, RPT)])


_sc_deg = pl.kernel(
    _sc_deg_body,
    out_type=jax.ShapeDtypeStruct((NC, NP, 16), jnp.float32),
    mesh=_mesh,
    scratch_types=[
        pltpu.VMEM_SHARED((NP, 16), jnp.float32),
        pltpu.VMEM((NB, 2, CH), jnp.int32),
        pltpu.VMEM((NB, 1, CH), jnp.float32),
        pltpu.VMEM((NB, CH, 16), jnp.float32),
    ] + [pltpu.SemaphoreType.DMA] * (2 * NB),
)


UN = 6


def _sc_layer_body(g_hbm, comb_hbm, w_hbm, out_hbm,
                   acc_sh, comb_v, w_v, rows_v, *sems):
    gsem = sems[0]
    ssem = sems[1]
    isem = sems[2:2 + UN]
    cid = lax.axis_index("c")
    sid = lax.axis_index("s")
    wid = sid * NC + cid

    _zero_rows(rows_v.at[0], CH, D)
    for r in range(RPT // CH):
        pltpu.sync_copy(rows_v.at[0], acc_sh.at[pl.ds(sid * RPT + r * CH, CH)])
    plsc.subcore_barrier()

    def scale(wslot, b):
        def group(g, _):
            wv = w_v[wslot, 0, pl.ds(g * 16, 16)]
            for l in range(16):
                wb = jnp.full((16,), wv[l], jnp.float32)
                j = g * 16 + l
                for q in range(D // 16):
                    sl = pl.ds(q * 16, 16)
                    rows_v[b, j, sl] = rows_v[b, j, sl] * wb
            return 0

        lax.fori_loop(0, CH // 16, group, 0)

    def sextet(s, _):
        cs = [UN * s + i for i in range(UN)]
        fills = []
        for i, c in enumerate(cs):
            fills.append(
                (pltpu.async_copy(comb_hbm.at[wid, c], comb_v.at[i], isem[i]),
                 pltpu.async_copy(w_hbm.at[wid, c], w_v.at[i], isem[i])))
        fills[0][0].wait()
        fills[0][1].wait()
        gd = pltpu.async_copy(g_hbm.at[comb_v.at[0, 0]], rows_v.at[0], gsem)
        gd.wait()
        sd = None
        for i in range(UN):
            if i + 1 < UN:
                fills[i + 1][0].wait()
                fills[i + 1][1].wait()
                gd = pltpu.async_copy(g_hbm.at[comb_v.at[i + 1, 0]],
                                      rows_v.at[(i + 1) % NB], gsem)
            scale(i, (i % NB))
            if sd is not None:
                sd.wait()
            sd = pltpu.async_copy(rows_v.at[i % NB],
                                  acc_sh.at[comb_v.at[i, 1]],
                                  ssem, add=True)
            if i + 1 < UN:
                gd.wait()
        sd.wait()
        return 0

    lax.fori_loop(0, NCHUNK // UN, sextet, 0)

    for c in range(NCHUNK - NCHUNK % UN, NCHUNK):
        pltpu.async_copy(comb_hbm.at[wid, c], comb_v.at[0], isem[0]).wait()
        pltpu.async_copy(w_hbm.at[wid, c], w_v.at[0], isem[0]).wait()
        pltpu.async_copy(g_hbm.at[comb_v.at[0, 0]], rows_v.at[0],
                         gsem).wait()
        scale(0, 0)
        pltpu.async_copy(rows_v.at[0], acc_sh.at[comb_v.at[0, 1]],
                         ssem, add=True).wait()

    plsc.subcore_barrier()
    pltpu.sync_copy(acc_sh.at[pl.ds(sid * RPT, RPT)],
                    out_hbm.at[cid, pl.ds(sid * RPT, RPT)])


_sc_layer = pl.kernel(
    _sc_layer_body,
    out_type=jax.ShapeDtypeStruct((NC, NP, D), jnp.float32),
    mesh=_mesh,
    scratch_types=[
        pltpu.VMEM_SHARED((NP, D), jnp.float32),
        pltpu.VMEM((UN, 2, CH), jnp.int32),
        pltpu.VMEM((UN, 1, CH), jnp.float32),
        pltpu.VMEM((NB, CH, D), jnp.float32),
    ] + [pltpu.SemaphoreType.DMA] * (2 + UN),
)


def _tc1_body(d_ref, x_ref, w_ref, a_ref, g_ref, dis_ref, dinv_ref):
    d = d_ref[...]
    deg = d[0, :, 0] + d[1, :, 0] + 1.0
    dis = lax.rsqrt(deg)
    dinv = 1.0 / deg
    a = jnp.dot(x_ref[...], w_ref[...], preferred_element_type=jnp.float32)
    a_ref[...] = a
    g_ref[...] = a * dis[:, None]
    dis_ref[...] = dis[:, None]
    dinv_ref[...] = dinv[:, None]


def _tc1(deg_parts, x, W1):
    return pl.pallas_call(
        _tc1_body,
        grid=(N // RB,),
        in_specs=[
            pl.BlockSpec((NC, RB, 16), lambda i: (0, i, 0)),
            pl.BlockSpec((RB, D), lambda i: (i, 0)),
            pl.BlockSpec((D, D), lambda i: (0, 0)),
        ],
        out_specs=[
            pl.BlockSpec((RB, D), lambda i: (i, 0)),
            pl.BlockSpec((RB, D), lambda i: (i, 0)),
            pl.BlockSpec((RB, 1), lambda i: (i, 0)),
            pl.BlockSpec((RB, 1), lambda i: (i, 0)),
        ],
        out_shape=[
            jax.ShapeDtypeStruct((N, D), jnp.float32),
            jax.ShapeDtypeStruct((N, D), jnp.float32),
            jax.ShapeDtypeStruct((N, 1), jnp.float32),
            jax.ShapeDtypeStruct((N, 1), jnp.float32),
        ],
    )(deg_parts, x, W1)


def _tc_mid_body(s_ref, a_ref, dis_ref, dinv_ref, b_ref, w_ref,
                 an_ref, gn_ref):
    s = s_ref[0] + s_ref[1]
    dis = dis_ref[...]
    z = dis * s + dinv_ref[...] * a_ref[...] + b_ref[...]
    x2 = jnp.maximum(z, 0.0)
    a2 = jnp.dot(x2, w_ref[...], preferred_element_type=jnp.float32)
    an_ref[...] = a2
    gn_ref[...] = a2 * dis


def _tc_mid(s, a_prev, dis, dinv, b, W):
    return pl.pallas_call(
        _tc_mid_body,
        grid=(N // RB,),
        in_specs=[
            pl.BlockSpec((NC, RB, D), lambda i: (0, i, 0)),
            pl.BlockSpec((RB, D), lambda i: (i, 0)),
            pl.BlockSpec((RB, 1), lambda i: (i, 0)),
            pl.BlockSpec((RB, 1), lambda i: (i, 0)),
            pl.BlockSpec((1, D), lambda i: (0, 0)),
            pl.BlockSpec((D, D), lambda i: (0, 0)),
        ],
        out_specs=[
            pl.BlockSpec((RB, D), lambda i: (i, 0)),
            pl.BlockSpec((RB, D), lambda i: (i, 0)),
        ],
        out_shape=[
            jax.ShapeDtypeStruct((N, D), jnp.float32),
            jax.ShapeDtypeStruct((N, D), jnp.float32),
        ],
    )(s, a_prev, dis, dinv, b, W)


def _tc_final_body(s_ref, a_ref, dis_ref, dinv_ref, b_ref, o_ref):
    s = s_ref[0] + s_ref[1]
    z = dis_ref[...] * s + dinv_ref[...] * a_ref[...] + b_ref[...]
    m = jnp.max(z, axis=1, keepdims=True)
    lse = jnp.log(jnp.sum(jnp.exp(z - m), axis=1, keepdims=True)) + m
    o_ref[...] = z - lse


def _tc_final(s, a_prev, dis, dinv, b):
    return pl.pallas_call(
        _tc_final_body,
        grid=(N // RB,),
        in_specs=[
            pl.BlockSpec((NC, RB, D), lambda i: (0, i, 0)),
            pl.BlockSpec((RB, D), lambda i: (i, 0)),
            pl.BlockSpec((RB, 1), lambda i: (i, 0)),
            pl.BlockSpec((RB, 1), lambda i: (i, 0)),
            pl.BlockSpec((1, D), lambda i: (0, 0)),
        ],
        out_specs=pl.BlockSpec((RB, D), lambda i: (i, 0)),
        out_shape=jax.ShapeDtypeStruct((N, D), jnp.float32),
    )(s, a_prev, dis, dinv, b)


@jax.jit
def kernel(x, edge_index, edge_attr, W1, b1, W2, b2, W3, b3):
    col = edge_index[1]
    w_t = edge_attr
    row_r = edge_index[0].reshape(NW, NCHUNK, 1, CH)
    col_r = col.reshape(NW, NCHUNK, 1, CH)
    comb = jnp.concatenate([row_r, col_r], axis=2)
    w_r = w_t.reshape(NW, NCHUNK, 1, CH)

    deg_parts = _sc_deg(comb, w_r)
    a1, g1, dis, dinv = _tc1(deg_parts, x, W1)
    s1 = _sc_layer(g1, comb, w_r)
    a2, g2 = _tc_mid(s1, a1, dis, dinv, b1.reshape(1, D), W2)
    s2 = _sc_layer(g2, comb, w_r)
    a3, g3 = _tc_mid(s2, a2, dis, dinv, b2.reshape(1, D), W3)
    s3 = _sc_layer(g3, comb, w_r)
    return _tc_final(s3, a3, dis, dinv, b3.reshape(1, D))
